# baseline (speedup 1.0000x reference)
import functools

import jax
import jax.numpy as jnp
from jax.experimental import pallas as pl

_BLOCK_ROWS = 4096


def _mm(a, b):
    return jnp.dot(a.astype(jnp.bfloat16), b.astype(jnp.bfloat16),
                   preferred_element_type=jnp.float32)


def _fused_mlp_body(x_ref, a0_ref, b0a_ref, b0_ref, b0b_ref, a1_ref,
                    b1pre_ref, p0_ref, p1_ref, b1post_ref, out_ref):
    x = x_ref[...]
    xa = x[:, 0:128]
    xb = x[:, 128:256]
    m0 = x[:, 256:257] != 0.0
    m1 = x[:, 257:258] != 0.0

    h = jnp.maximum(_mm(xa, a0_ref[...]) + b0a_ref[...], 0.0)
    x0 = jnp.maximum(_mm(h, b0_ref[...]) + b0b_ref[...], 0.0)
    x0 = jnp.where(m0, x0, 0.0)

    h1 = jnp.maximum(_mm(xb, a1_ref[...]) + b1pre_ref[...], 0.0)

    out = _mm(x0, p0_ref[...]) + _mm(h1, p1_ref[...]) + b1post_ref[...]
    out_ref[...] = jnp.where(m1, out, 0.0)


@jax.jit
def kernel(input_val, W0a, b0a, W0b, b0b, W1pre, b1pre, W1post, b1post):
    n, d_in = input_val.shape
    block = min(_BLOCK_ROWS, n)
    grid = (n // block,)

    a0 = W0a.T
    b0 = W0b.T
    a1 = W1pre.T
    p0 = W1post[:, :128].T
    p1 = W1post[:, 128:].T

    row_spec = lambda w: pl.BlockSpec(w.shape, lambda i: (0, 0))
    out = pl.pallas_call(
        _fused_mlp_body,
        grid=grid,
        in_specs=[
            pl.BlockSpec((block, d_in), lambda i: (i, 0)),
            row_spec(a0),
            pl.BlockSpec((1, 256), lambda i: (0, 0)),
            row_spec(b0),
            pl.BlockSpec((1, 128), lambda i: (0, 0)),
            row_spec(a1),
            pl.BlockSpec((1, 256), lambda i: (0, 0)),
            row_spec(p0),
            row_spec(p1),
            pl.BlockSpec((1, 128), lambda i: (0, 0)),
        ],
        out_specs=pl.BlockSpec((block, 128), lambda i: (i, 0)),
        out_shape=jax.ShapeDtypeStruct((n, 128), input_val.dtype),
    )(input_val, a0, b0a.reshape(1, 256), b0, b0b.reshape(1, 128),
      a1, b1pre.reshape(1, 256), p0, p1, b1post.reshape(1, 128))
    return out


# speedup vs baseline: 1.2199x; 1.2199x over previous
import jax
import jax.numpy as jnp
from jax import lax
from jax.experimental import pallas as pl

_BLOCK_ROWS = 4096

_DN_T = (((1,), (1,)), ((), ()))


def _mmt(x, w):
    return lax.dot_general(x.astype(jnp.bfloat16), w.astype(jnp.bfloat16),
                           _DN_T, preferred_element_type=jnp.float32)


def _fused_mlp_body(x_ref, w0a_ref, b0a_ref, w0b_ref, b0b_ref, w1pre_ref,
                    b1pre_ref, w1post_ref, b1post_ref, out_ref):
    x = x_ref[...]
    xa = x[:, 0:128]
    xb = x[:, 128:256]
    m0 = x[:, 256:257] != 0.0
    m1 = x[:, 257:258] != 0.0

    h = jnp.maximum(_mmt(xa, w0a_ref[...]) + b0a_ref[...], 0.0)
    x0 = jnp.maximum(_mmt(h, w0b_ref[...]) + b0b_ref[...], 0.0)
    x0 = jnp.where(m0, x0, 0.0)

    h1 = jnp.maximum(_mmt(xb, w1pre_ref[...]) + b1pre_ref[...], 0.0)

    out = (_mmt(x0, w1post_ref[:, 0:128]) + _mmt(h1, w1post_ref[:, 128:384])
           + b1post_ref[...])
    out_ref[...] = jnp.where(m1, out, 0.0)


@jax.jit
def kernel(input_val, W0a, b0a, W0b, b0b, W1pre, b1pre, W1post, b1post):
    n, d_in = input_val.shape
    block = min(_BLOCK_ROWS, n)
    grid = (n // block,)

    full = lambda w: pl.BlockSpec(w.shape, lambda i: (0,) * w.ndim)
    b0a2 = b0a.reshape(1, 256)
    b0b2 = b0b.reshape(1, 128)
    b1pre2 = b1pre.reshape(1, 256)
    b1post2 = b1post.reshape(1, 128)
    out = pl.pallas_call(
        _fused_mlp_body,
        grid=grid,
        in_specs=[
            pl.BlockSpec((block, d_in), lambda i: (i, 0)),
            full(W0a), full(b0a2), full(W0b), full(b0b2),
            full(W1pre), full(b1pre2), full(W1post), full(b1post2),
        ],
        out_specs=pl.BlockSpec((block, 128), lambda i: (i, 0)),
        out_shape=jax.ShapeDtypeStruct((n, 128), input_val.dtype),
    )(input_val, W0a, b0a2, W0b, b0b2, W1pre, b1pre2, W1post, b1post2)
    return out
